# Initial kernel scaffold; baseline (speedup 1.0000x reference)
import jax
import jax.numpy as jnp
from jax.experimental import pallas as pl


def kernel(node_input, att_input, adj, Fadj, feat, params):
    raise NotImplementedError("write your pallas kernel here")


# speedup vs baseline: 1.2454x; 1.2454x over previous
import jax
import jax.numpy as jnp
from jax.experimental import pallas as pl
from jax.experimental.pallas import tpu as pltpu

N = 4096
B = 2
NFEAT = 128
NHID = 64
NCLASS = 8
NATTR = 128
EPS = 1e-5

ROWS = 512
NBLK = N // ROWS


def _bn2(x, g, be):
    mu = jnp.mean(x, axis=0, keepdims=True)
    var = jnp.mean((x - mu) * (x - mu), axis=0, keepdims=True)
    return (x - mu) * jax.lax.rsqrt(var + EPS) * g + be


def _bn3(x, g, be):
    mu = jnp.mean(x, axis=(0, 1), keepdims=True)
    var = jnp.mean((x - mu) * (x - mu), axis=(0, 1), keepdims=True)
    return (x - mu) * jax.lax.rsqrt(var + EPS) * g + be


def _mm(a, b):
    return jnp.dot(a, b, preferred_element_type=jnp.float32)


def _pass_body(adj_ref, r_ref, p_ref):
    p_ref[...] = _mm(adj_ref[...], r_ref[...])


def _adj_pass(adj, r):
    k = r.shape[1]
    return pl.pallas_call(
        _pass_body,
        grid=(NBLK,),
        in_specs=[
            pl.BlockSpec((ROWS, N), lambda i: (i, 0)),
            pl.BlockSpec((N, k), lambda i: (0, 0)),
        ],
        out_specs=pl.BlockSpec((ROWS, k), lambda i: (i, 0)),
        out_shape=jax.ShapeDtypeStruct((N, k), jnp.float32),
    )(adj, r)


def _pre_body(feat_ref, node_ref, att_ref, fadj_ref,
              wge1_ref, wse1_ref, wsge1_ref, wsse1_ref, wae1_ref,
              b1g_ref, b1s_ref, bae1_ref,
              r1_ref, c1_ref, c2_ref, m3_ref):
    feat = feat_ref[...]
    fadj = fadj_ref[...]
    r1_ref[:, 0:NHID] = _mm(feat, wge1_ref[...])
    c1_ref[...] = _mm(feat, wsge1_ref[...]) + b1g_ref[...]
    wse1 = wse1_ref[...]
    wsse1 = wsse1_ref[...]
    wae1 = wae1_ref[...]
    for b in range(B):
        x = node_ref[b]
        r1_ref[:, NHID * (b + 1):NHID * (b + 2)] = (
            x[:, 0:1] * wse1[0:1, :] + x[:, 1:2] * wse1[1:2, :])
        c2_ref[b] = (x[:, 0:1] * wsse1[0:1, :] + x[:, 1:2] * wsse1[1:2, :]
                     + b1s_ref[...])
        aw = att_ref[b][:, 0:1] * wae1[0:1, :]
        m3_ref[b] = _mm(fadj, aw) + bae1_ref[...]


def _g1_body(p1_ref, c1_ref, c2_ref, m3_ref, fadj_ref, att_ref,
             wcnd1_ref, bcnd1_ref, gbn1_ref, bebn1_ref,
             gbnge1_ref, bebnge1_ref,
             wge2_ref, wse2_ref, wsge2_ref, wsse2_ref, b2g_ref, b2s_ref,
             wsae1_ref, bsae1_ref, gbnae1_ref, bebnae1_ref,
             wae2_ref, bae2_ref,
             model_ref, r2_ref, c12_ref, c22_ref, m32_ref, ae_ref):
    p1 = p1_ref[...]
    m1 = p1[:, 0:NHID] + c1_ref[...]
    fadj = fadj_ref[...]
    wcnd1 = wcnd1_ref[...]
    ms = []
    for b in range(B):
        m2 = p1[:, NHID * (b + 1):NHID * (b + 2)] + c2_ref[b]
        cc = jnp.concatenate([m1, m2, m3_ref[b]], axis=1)
        ms.append(_mm(cc, wcnd1) + bcnd1_ref[...])
    mcat = jnp.stack(ms)
    model = jax.nn.relu(_bn3(mcat, gbn1_ref[...], bebn1_ref[...]))
    model_ref[...] = model
    g1 = jax.nn.relu(_bn2(m1, gbnge1_ref[...], bebnge1_ref[...]))
    r2_ref[:, 0:NHID] = _mm(g1, wge2_ref[...])
    c12_ref[...] = _mm(g1, wsge2_ref[...]) + b2g_ref[...]
    wse2 = wse2_ref[...]
    wsse2 = wsse2_ref[...]
    wsae1 = wsae1_ref[...]
    t3s = []
    for b in range(B):
        mb = model[b]
        r2_ref[:, NHID * (b + 1):NHID * (b + 2)] = _mm(mb, wse2)
        c22_ref[b] = _mm(mb, wsse2) + b2s_ref[...]
        ft = jax.lax.dot_general(fadj, mb, (((0,), (0,)), ((), ())),
                                 preferred_element_type=jnp.float32)
        t3s.append(ft + att_ref[b][:, 0:1] * wsae1[0:1, :] + bsae1_ref[...])
    t3 = jnp.stack(t3s)
    ae = jax.nn.relu(_bn3(t3, gbnae1_ref[...], bebnae1_ref[...]))
    ae_ref[...] = ae
    wae2 = wae2_ref[...]
    for b in range(B):
        m32_ref[b] = _mm(fadj, _mm(ae[b], wae2)) + bae2_ref[...]


def _g2_body(p2_ref, c12_ref, c22_ref, m32_ref, ae_ref, fadj_ref,
             wcnd2_ref, bcnd2_ref, gbn2_ref, bebn2_ref,
             gbnge2_ref, bebnge2_ref,
             wsae2_ref, bsae2_ref, gbnae2_ref, bebnae2_ref,
             wge3c_ref, wse3c_ref, wsge3c_ref, wsse3c_ref, wae3c_ref,
             cvec3_ref,
             r3_ref, a_ref):
    p2 = p2_ref[...]
    m1 = p2[:, 0:NHID] + c12_ref[...]
    fadj = fadj_ref[...]
    wcnd2 = wcnd2_ref[...]
    ms = []
    for b in range(B):
        m2 = p2[:, NHID * (b + 1):NHID * (b + 2)] + c22_ref[b]
        cc = jnp.concatenate([m1, m2, m32_ref[b]], axis=1)
        ms.append(_mm(cc, wcnd2) + bcnd2_ref[...])
    mcat = jnp.stack(ms)
    model = jax.nn.relu(_bn3(mcat, gbn2_ref[...], bebn2_ref[...]))
    g2 = jax.nn.relu(_bn2(m1, gbnge2_ref[...], bebnge2_ref[...]))
    wsae2 = wsae2_ref[...]
    t3s = []
    for b in range(B):
        ft = jax.lax.dot_general(fadj, model[b], (((0,), (0,)), ((), ())),
                                 preferred_element_type=jnp.float32)
        t3s.append(ft + _mm(ae_ref[b], wsae2) + bsae2_ref[...])
    t3 = jnp.stack(t3s)
    u = jax.nn.relu(_bn3(t3, gbnae2_ref[...], bebnae2_ref[...]))
    g2ge = _mm(g2, wge3c_ref[...])
    g2sge = _mm(g2, wsge3c_ref[...])
    for b in range(B):
        r3_ref[:, NCLASS * b:NCLASS * (b + 1)] = g2ge + _mm(model[b],
                                                            wse3c_ref[...])
        a_ref[b] = (g2sge + _mm(model[b], wsse3c_ref[...])
                    + _mm(fadj, _mm(u[b], wae3c_ref[...]))
                    + cvec3_ref[...])


def _pass3_body(adj_ref, r3_ref, a_ref, out_ref):
    p = _mm(adj_ref[...], r3_ref[...])
    for b in range(B):
        out_ref[b] = jax.nn.sigmoid(
            p[:, NCLASS * b:NCLASS * (b + 1)] + a_ref[b])


def _adj_pass3(adj, r3, a):
    return pl.pallas_call(
        _pass3_body,
        grid=(NBLK,),
        in_specs=[
            pl.BlockSpec((ROWS, N), lambda i: (i, 0)),
            pl.BlockSpec((N, B * NCLASS), lambda i: (0, 0)),
            pl.BlockSpec((B, ROWS, NCLASS), lambda i: (0, i, 0)),
        ],
        out_specs=pl.BlockSpec((B, ROWS, NCLASS), lambda i: (0, i, 0)),
        out_shape=jax.ShapeDtypeStruct((B, N, NCLASS), jnp.float32),
    )(adj, r3, a)


def kernel(node_input, att_input, adj, Fadj, feat, params):
    p = params
    r = lambda v: v.reshape(1, -1)

    b1g = r(p["b_ge1"] + p["b_sge1"])
    b1s = r(p["b_se1"] + p["b_sse1"])
    b2g = r(p["b_ge2"] + p["b_sge2"])
    b2s = r(p["b_se2"] + p["b_sse2"])
    wc3 = p["W_cnd3"]
    wge3c = p["W_ge3"] @ wc3[0:NCLASS]
    wse3c = p["W_se3"] @ wc3[NCLASS:2 * NCLASS]
    wsge3c = p["W_sge3"] @ wc3[0:NCLASS]
    wsse3c = p["W_sse3"] @ wc3[NCLASS:2 * NCLASS]
    wae3c = p["W_ae3"] @ wc3[2 * NCLASS:]
    cvec3 = r((p["b_ge3"] + p["b_sge3"]) @ wc3[0:NCLASS]
              + (p["b_se3"] + p["b_sse3"]) @ wc3[NCLASS:2 * NCLASS]
              + p["b_ae3"] @ wc3[2 * NCLASS:] + p["b_cnd3"])

    f32 = jnp.float32
    sd = jax.ShapeDtypeStruct

    r1, c1, c2, m3 = pl.pallas_call(
        _pre_body,
        out_shape=[sd((N, 3 * NHID), f32), sd((N, NHID), f32),
                   sd((B, N, NHID), f32), sd((B, N, NHID), f32)],
    )(feat, node_input, att_input, Fadj,
      p["W_ge1"], p["W_se1"], p["W_sge1"], p["W_sse1"], p["W_ae1"],
      b1g, b1s, r(p["b_ae1"]))

    p1 = _adj_pass(adj, r1)

    model, r2, c12, c22, m32, ae = pl.pallas_call(
        _g1_body,
        out_shape=[sd((B, N, NHID), f32), sd((N, 3 * NHID), f32),
                   sd((N, NHID), f32), sd((B, N, NHID), f32),
                   sd((B, N, NHID), f32), sd((B, NATTR, NHID), f32)],
    )(p1, c1, c2, m3, Fadj, att_input,
      p["W_cnd1"], r(p["b_cnd1"]), r(p["g_bn1"]), r(p["be_bn1"]),
      r(p["g_bn_ge1"]), r(p["be_bn_ge1"]),
      p["W_ge2"], p["W_se2"], p["W_sge2"], p["W_sse2"], b2g, b2s,
      p["W_sae1"], r(p["b_sae1"]), r(p["g_bn_ae1"]), r(p["be_bn_ae1"]),
      p["W_ae2"], r(p["b_ae2"]))

    p2 = _adj_pass(adj, r2)

    r3, a = pl.pallas_call(
        _g2_body,
        out_shape=[sd((N, B * NCLASS), f32), sd((B, N, NCLASS), f32)],
    )(p2, c12, c22, m32, ae, Fadj,
      p["W_cnd2"], r(p["b_cnd2"]), r(p["g_bn2"]), r(p["be_bn2"]),
      r(p["g_bn_ge2"]), r(p["be_bn_ge2"]),
      p["W_sae2"], r(p["b_sae2"]), r(p["g_bn_ae2"]), r(p["be_bn_ae2"]),
      wge3c, wse3c, wsge3c, wsse3c, wae3c, cvec3)

    return _adj_pass3(adj, r3, a)
